# baseline (speedup 1.0000x reference)
import functools

import jax
import jax.numpy as jnp
from jax import lax
from jax.experimental import pallas as pl
from jax.experimental.pallas import tpu as pltpu
from jax.experimental.pallas import tpu_sc as plsc

D = 16
NC = 2
NS = 16
NW = NC * NS
CHUNK = 125
ALPHA = 1.0 / (D ** 0.5)


def _mesh():
    return plsc.VectorSubcoreMesh(
        core_axis_name="c", subcore_axis_name="s", num_cores=NC,
        num_subcores=NS)


def _make_gather(ep, n_nodes):
    e_per_w = ep // NW
    ch_per_w = e_per_w // CHUNK

    @functools.partial(
        pl.kernel, mesh=_mesh(),
        compiler_params=pltpu.CompilerParams(use_tc_tiling_on_sc=False),
        out_type=jax.ShapeDtypeStruct((ep, D), jnp.float32),
        scratch_types=[
            pltpu.VMEM((ch_per_w, CHUNK), jnp.int32),
            pltpu.VMEM((e_per_w, D), jnp.float32),
            pltpu.SemaphoreType.DMA,
        ],
    )
    def gather_kernel(idx_hbm, table_hbm, x_hbm, idx_v, rows_v, sem):
        wid = lax.axis_index("s") * NC + lax.axis_index("c")
        pltpu.sync_copy(idx_hbm.at[pl.ds(wid * ch_per_w, ch_per_w)], idx_v)

        def fire(j, carry):
            pltpu.async_copy(
                table_hbm.at[idx_v.at[j]],
                rows_v.at[pl.ds(j * CHUNK, CHUNK)], sem)
            return carry

        lax.fori_loop(0, ch_per_w, fire, 0)
        pltpu.make_async_copy(
            x_hbm.at[pl.ds(wid * e_per_w, e_per_w)], rows_v, sem).wait()
        pltpu.sync_copy(rows_v, x_hbm.at[pl.ds(wid * e_per_w, e_per_w)])

    return gather_kernel


def _make_scatter(ep, n_pad):
    e_per_w = ep // NW
    ch_per_w = e_per_w // CHUNK
    zrows = n_pad // NS

    @functools.partial(
        pl.kernel, mesh=_mesh(),
        compiler_params=pltpu.CompilerParams(use_tc_tiling_on_sc=False),
        out_type=(jax.ShapeDtypeStruct((NC, n_pad, D), jnp.float32),
                  jax.ShapeDtypeStruct((NC, n_pad), jnp.float32)),
        scratch_types=[
            pltpu.VMEM((ch_per_w, CHUNK), jnp.int32),
            pltpu.VMEM((e_per_w, D), jnp.float32),
            pltpu.VMEM((zrows, D), jnp.float32),
            pltpu.VMEM((zrows,), jnp.float32),
            pltpu.VMEM((((CHUNK + 15) // 16) * 16,), jnp.float32),
            pltpu.VMEM_SHARED((n_pad, D), jnp.float32),
            pltpu.VMEM_SHARED((n_pad,), jnp.float32),
        ],
    )
    def scatter_kernel(idx_hbm, tp_hbm, psum_hbm, pcnt_hbm,
                       idx_v, rows_v, zbuf, zbufc, ones_v, acc_sh, cnt_sh):
        cid = lax.axis_index("c")
        sid = lax.axis_index("s")
        wid = sid * NC + cid
        zero16 = jnp.zeros((16,), jnp.float32)
        one16 = jnp.ones((16,), jnp.float32)

        def zrow(i, carry):
            zbuf[i, :] = zero16
            return carry

        lax.fori_loop(0, zrows, zrow, 0)

        def zcnt(i, carry):
            zbufc[pl.ds(i * 16, 16)] = zero16
            return carry

        lax.fori_loop(0, zrows // 16, zcnt, 0)

        def orow(i, carry):
            ones_v[pl.ds(i * 16, 16)] = one16
            return carry

        lax.fori_loop(0, (CHUNK + 15) // 16, orow, 0)

        pltpu.sync_copy(zbuf, acc_sh.at[pl.ds(sid * zrows, zrows)])
        pltpu.sync_copy(zbufc, cnt_sh.at[pl.ds(sid * zrows, zrows)])
        plsc.subcore_barrier()

        pltpu.sync_copy(idx_hbm.at[pl.ds(wid * ch_per_w, ch_per_w)], idx_v)
        pltpu.sync_copy(tp_hbm.at[pl.ds(wid * e_per_w, e_per_w)], rows_v)

        def body(j, carry):
            pltpu.sync_copy(rows_v.at[pl.ds(j * CHUNK, CHUNK)],
                            acc_sh.at[idx_v.at[j]], add=True)
            pltpu.sync_copy(ones_v.at[pl.ds(0, CHUNK)],
                            cnt_sh.at[idx_v.at[j]], add=True)
            return carry

        lax.fori_loop(0, ch_per_w, body, 0)
        plsc.subcore_barrier()

        pltpu.sync_copy(acc_sh.at[pl.ds(sid * zrows, zrows)],
                        psum_hbm.at[cid].at[pl.ds(sid * zrows, zrows)])
        pltpu.sync_copy(cnt_sh.at[pl.ds(sid * zrows, zrows)],
                        pcnt_hbm.at[cid].at[pl.ds(sid * zrows, zrows)])

    return scatter_kernel


def _run_tp(ea_pk, x_pk, sh_pk, w1blk, b1_pk, w2cat, scat, b2_pk, gb=400):
    gtot = ea_pk.shape[0]
    wn2 = w2cat.shape[1]
    grid = (gtot // gb,)

    def body(ea_ref, x_ref, sh_ref, w1_ref, b1_ref, w2_ref, sc_ref, b2_ref,
             tp_ref):
        h = jnp.maximum(
            jnp.dot(ea_ref[...], w1_ref[...],
                    preferred_element_type=jnp.float32) + b1_ref[...], 0.0)
        xs = x_ref[...] * sh_ref[...] * ALPHA
        tpw = jnp.dot(h.astype(jnp.bfloat16), w2_ref[...],
                      preferred_element_type=jnp.float32) + b2_ref[...]
        xse = jnp.dot(xs.astype(jnp.bfloat16), sc_ref[...],
                      preferred_element_type=jnp.float32)
        z = tpw * xse
        acc = z[:, 0:128]
        for i in range(1, D):
            acc = acc + z[:, i * 128:(i + 1) * 128]
        tp_ref[...] = acc

    return pl.pallas_call(
        body,
        grid=grid,
        in_specs=[
            pl.BlockSpec((gb, 128), lambda i: (i, 0)),
            pl.BlockSpec((gb, 128), lambda i: (i, 0)),
            pl.BlockSpec((gb, 128), lambda i: (i, 0)),
            pl.BlockSpec((128, 128), lambda i: (0, 0)),
            pl.BlockSpec((1, 128), lambda i: (0, 0)),
            pl.BlockSpec((128, wn2), lambda i: (0, 0)),
            pl.BlockSpec((128, wn2), lambda i: (0, 0)),
            pl.BlockSpec((1, wn2), lambda i: (0, 0)),
        ],
        out_specs=pl.BlockSpec((gb, 128), lambda i: (i, 0)),
        out_shape=jax.ShapeDtypeStruct((gtot, 128), jnp.float32),
    )(ea_pk, x_pk, sh_pk, w1blk, b1_pk, w2cat, scat, b2_pk)


def _make_combine(n_pad):
    rows_w = n_pad // NW

    @functools.partial(
        pl.kernel, mesh=_mesh(),
        compiler_params=pltpu.CompilerParams(
            use_tc_tiling_on_sc=False, needs_layout_passes=False),
        out_type=jax.ShapeDtypeStruct((n_pad, D), jnp.float32),
        scratch_types=[
            pltpu.VMEM((rows_w, D), jnp.float32),
            pltpu.VMEM((rows_w, D), jnp.float32),
            pltpu.VMEM((rows_w,), jnp.float32),
            pltpu.VMEM((rows_w,), jnp.float32),
            pltpu.VMEM((rows_w,), jnp.float32),
            pltpu.VMEM((rows_w, D), jnp.float32),
            pltpu.VMEM((rows_w, D), jnp.float32),
        ],
    )
    def combine_kernel(psum_hbm, pcnt_hbm, na_hbm, out_hbm,
                       p0v, p1v, c0v, c1v, invv, nav, outv):
        wid = lax.axis_index("s") * NC + lax.axis_index("c")
        base = wid * rows_w
        pltpu.sync_copy(psum_hbm.at[0].at[pl.ds(base, rows_w)], p0v)
        pltpu.sync_copy(psum_hbm.at[1].at[pl.ds(base, rows_w)], p1v)
        pltpu.sync_copy(pcnt_hbm.at[0].at[pl.ds(base, rows_w)], c0v)
        pltpu.sync_copy(pcnt_hbm.at[1].at[pl.ds(base, rows_w)], c1v)
        pltpu.sync_copy(na_hbm.at[pl.ds(base, rows_w)], nav)

        def grp(g, carry):
            c = c0v[pl.ds(g * 16, 16)] + c1v[pl.ds(g * 16, 16)]
            invv[pl.ds(g * 16, 16)] = 1.0 / jnp.maximum(c, 1.0)
            return carry

        lax.fori_loop(0, rows_w // 16, grp, 0)

        def row(r, carry):
            iv = plsc.load_gather(invv, [jnp.full((16,), r, jnp.int32)])
            outv[r, :] = (p0v[r, :] + p1v[r, :]) * iv + nav[r, :]
            return carry

        lax.fori_loop(0, rows_w, row, 0)
        pltpu.sync_copy(outv, out_hbm.at[pl.ds(base, rows_w)])

    return combine_kernel


def kernel(node_attr, edge_index, edge_attr, edge_sh, W1, b1, W2, b2):
    n_nodes, d = node_attr.shape
    e = edge_attr.shape[0]
    n_pad = ((n_nodes + 1 + NW * D - 1) // (NW * D)) * (NW * D)

    src2 = edge_index[0].astype(jnp.int32).reshape(e // CHUNK, CHUNK)
    dst2 = edge_index[1].astype(jnp.int32).reshape(e // CHUNK, CHUNK)
    na_pad = jnp.concatenate(
        [node_attr, jnp.zeros((n_pad - n_nodes, d), jnp.float32)])

    eye8 = jnp.eye(8, dtype=jnp.float32)
    w1blk = jnp.kron(eye8, W1)
    b1_pk = jnp.tile(b1, 8)[None, :]
    w2r = W2.reshape(D, D, D)
    w2cat = jnp.concatenate(
        [jnp.kron(eye8, w2r[:, i, :]) for i in range(D)],
        axis=1).astype(jnp.bfloat16)
    sel = [jnp.kron(eye8, jnp.zeros((D, D)).at[i].set(1.0))
           for i in range(D)]
    scat = jnp.concatenate(sel, axis=1).astype(jnp.bfloat16)
    b2_pk = jnp.tile(b2.reshape(D, 1, D), (1, 8, 1)).reshape(1, 8 * D * D)

    ea_pk = edge_attr.reshape(e // 8, 8 * D)
    sh_pk = jnp.broadcast_to(edge_sh, (e, D)).reshape(e // 8, 8 * D)

    x = _make_gather(e, n_nodes)(dst2, node_attr)
    tp_pk = _run_tp(ea_pk, x.reshape(e // 8, 8 * D), sh_pk,
                    w1blk, b1_pk, w2cat, scat, b2_pk)
    psum, pcnt = _make_scatter(e, n_pad)(src2, tp_pk.reshape(e, D))
    out_pad = _make_combine(n_pad)(psum, pcnt, na_pad)
    return out_pad[:n_nodes]


# speedup vs baseline: 1.2316x; 1.2316x over previous
import functools

import jax
import jax.numpy as jnp
from jax import lax
from jax.experimental import pallas as pl
from jax.experimental.pallas import tpu as pltpu
from jax.experimental.pallas import tpu_sc as plsc

D = 16
NC = 2
NS = 16
NW = NC * NS
CHUNK = 125
ALPHA = 1.0 / (D ** 0.5)


def _mesh():
    return plsc.VectorSubcoreMesh(
        core_axis_name="c", subcore_axis_name="s", num_cores=NC,
        num_subcores=NS)


def _make_gather(ep, n_nodes):
    e_per_w = ep // NW
    ch_per_w = e_per_w // CHUNK

    @functools.partial(
        pl.kernel, mesh=_mesh(),
        compiler_params=pltpu.CompilerParams(use_tc_tiling_on_sc=False),
        out_type=jax.ShapeDtypeStruct((ep, D), jnp.float32),
        scratch_types=[
            pltpu.VMEM((ch_per_w, CHUNK), jnp.int32),
            pltpu.VMEM((e_per_w, D), jnp.float32),
            pltpu.SemaphoreType.DMA,
        ],
    )
    def gather_kernel(idx_hbm, table_hbm, x_hbm, idx_v, rows_v, sem):
        wid = lax.axis_index("s") * NC + lax.axis_index("c")
        pltpu.sync_copy(idx_hbm.at[pl.ds(wid * ch_per_w, ch_per_w)], idx_v)

        def fire(j, carry):
            pltpu.async_copy(
                table_hbm.at[idx_v.at[j]],
                rows_v.at[pl.ds(j * CHUNK, CHUNK)], sem)
            return carry

        lax.fori_loop(0, ch_per_w, fire, 0)
        pltpu.make_async_copy(
            x_hbm.at[pl.ds(wid * e_per_w, e_per_w)], rows_v, sem).wait()
        pltpu.sync_copy(rows_v, x_hbm.at[pl.ds(wid * e_per_w, e_per_w)])

    return gather_kernel


def _make_scatter(ep, n_pad):
    e_per_w = ep // NW
    ch_per_w = e_per_w // CHUNK
    zrows = n_pad // NS

    @functools.partial(
        pl.kernel, mesh=_mesh(),
        compiler_params=pltpu.CompilerParams(use_tc_tiling_on_sc=False),
        out_type=(jax.ShapeDtypeStruct((NC, n_pad, D), jnp.float32),
                  jax.ShapeDtypeStruct((NC, n_pad), jnp.float32)),
        scratch_types=[
            pltpu.VMEM((ch_per_w, CHUNK), jnp.int32),
            pltpu.VMEM((e_per_w, D), jnp.float32),
            pltpu.VMEM((zrows, D), jnp.float32),
            pltpu.VMEM((zrows,), jnp.float32),
            pltpu.VMEM((((CHUNK + 15) // 16) * 16,), jnp.float32),
            pltpu.VMEM_SHARED((n_pad, D), jnp.float32),
            pltpu.VMEM_SHARED((n_pad,), jnp.float32),
        ],
    )
    def scatter_kernel(idx_hbm, tp_hbm, psum_hbm, pcnt_hbm,
                       idx_v, rows_v, zbuf, zbufc, ones_v, acc_sh, cnt_sh):
        cid = lax.axis_index("c")
        sid = lax.axis_index("s")
        wid = sid * NC + cid
        zero16 = jnp.zeros((16,), jnp.float32)
        one16 = jnp.ones((16,), jnp.float32)

        def zrow(i, carry):
            zbuf[i, :] = zero16
            return carry

        lax.fori_loop(0, zrows, zrow, 0)

        def zcnt(i, carry):
            zbufc[pl.ds(i * 16, 16)] = zero16
            return carry

        lax.fori_loop(0, zrows // 16, zcnt, 0)

        def orow(i, carry):
            ones_v[pl.ds(i * 16, 16)] = one16
            return carry

        lax.fori_loop(0, (CHUNK + 15) // 16, orow, 0)

        pltpu.sync_copy(zbuf, acc_sh.at[pl.ds(sid * zrows, zrows)])
        pltpu.sync_copy(zbufc, cnt_sh.at[pl.ds(sid * zrows, zrows)])
        plsc.subcore_barrier()

        pltpu.sync_copy(idx_hbm.at[pl.ds(wid * ch_per_w, ch_per_w)], idx_v)
        pltpu.sync_copy(tp_hbm.at[pl.ds(wid * e_per_w, e_per_w)], rows_v)

        def body(j, carry):
            pltpu.sync_copy(rows_v.at[pl.ds(j * CHUNK, CHUNK)],
                            acc_sh.at[idx_v.at[j]], add=True)
            pltpu.sync_copy(ones_v.at[pl.ds(0, CHUNK)],
                            cnt_sh.at[idx_v.at[j]], add=True)
            return carry

        lax.fori_loop(0, ch_per_w, body, 0)
        plsc.subcore_barrier()

        pltpu.sync_copy(acc_sh.at[pl.ds(sid * zrows, zrows)],
                        psum_hbm.at[cid].at[pl.ds(sid * zrows, zrows)])
        pltpu.sync_copy(cnt_sh.at[pl.ds(sid * zrows, zrows)],
                        pcnt_hbm.at[cid].at[pl.ds(sid * zrows, zrows)])

    return scatter_kernel


def _tp_body(ea_ref, x_ref, sh_ref, w1t_ref, b1t_ref, w2t_ref, b2t_ref,
             tp_ref):
    eaT = ea_ref[...]
    xsT = jnp.transpose(x_ref[...]) * sh_ref[...] * ALPHA
    hT = jnp.maximum(
        jnp.dot(w1t_ref[...], eaT,
                preferred_element_type=jnp.float32) + b1t_ref[...], 0.0)
    tpwT = jnp.dot(w2t_ref[...], hT.astype(jnp.bfloat16),
                   preferred_element_type=jnp.float32) + b2t_ref[...]
    acc = xsT[0:1, :] * tpwT[0:D, :]
    for i in range(1, D):
        acc = acc + xsT[i:i + 1, :] * tpwT[i * D:(i + 1) * D, :]
    tp_ref[...] = jnp.transpose(acc)


def _run_tp(eaT, x, shT, w1t, b1t, w2t, b2t, blk=3200):
    f, ep = eaT.shape
    wn = w2t.shape[0]
    grid = (ep // blk,)
    return pl.pallas_call(
        _tp_body,
        grid=grid,
        in_specs=[
            pl.BlockSpec((f, blk), lambda i: (0, i)),
            pl.BlockSpec((blk, D), lambda i: (i, 0)),
            pl.BlockSpec((1, blk), lambda i: (0, i)),
            pl.BlockSpec((f, f), lambda i: (0, 0)),
            pl.BlockSpec((f, 1), lambda i: (0, 0)),
            pl.BlockSpec((wn, f), lambda i: (0, 0)),
            pl.BlockSpec((wn, 1), lambda i: (0, 0)),
        ],
        out_specs=pl.BlockSpec((blk, D), lambda i: (i, 0)),
        out_shape=jax.ShapeDtypeStruct((ep, D), jnp.float32),
    )(eaT, x, shT, w1t, b1t, w2t, b2t)


def _make_combine(n_pad):
    rows_w = n_pad // NW

    @functools.partial(
        pl.kernel, mesh=_mesh(),
        compiler_params=pltpu.CompilerParams(
            use_tc_tiling_on_sc=False, needs_layout_passes=False),
        out_type=jax.ShapeDtypeStruct((n_pad, D), jnp.float32),
        scratch_types=[
            pltpu.VMEM((rows_w, D), jnp.float32),
            pltpu.VMEM((rows_w, D), jnp.float32),
            pltpu.VMEM((rows_w,), jnp.float32),
            pltpu.VMEM((rows_w,), jnp.float32),
            pltpu.VMEM((rows_w,), jnp.float32),
            pltpu.VMEM((rows_w, D), jnp.float32),
            pltpu.VMEM((rows_w, D), jnp.float32),
        ],
    )
    def combine_kernel(psum_hbm, pcnt_hbm, na_hbm, out_hbm,
                       p0v, p1v, c0v, c1v, invv, nav, outv):
        wid = lax.axis_index("s") * NC + lax.axis_index("c")
        base = wid * rows_w
        pltpu.sync_copy(psum_hbm.at[0].at[pl.ds(base, rows_w)], p0v)
        pltpu.sync_copy(psum_hbm.at[1].at[pl.ds(base, rows_w)], p1v)
        pltpu.sync_copy(pcnt_hbm.at[0].at[pl.ds(base, rows_w)], c0v)
        pltpu.sync_copy(pcnt_hbm.at[1].at[pl.ds(base, rows_w)], c1v)
        pltpu.sync_copy(na_hbm.at[pl.ds(base, rows_w)], nav)

        def grp(g, carry):
            c = c0v[pl.ds(g * 16, 16)] + c1v[pl.ds(g * 16, 16)]
            invv[pl.ds(g * 16, 16)] = 1.0 / jnp.maximum(c, 1.0)
            return carry

        lax.fori_loop(0, rows_w // 16, grp, 0)

        def row(r, carry):
            iv = plsc.load_gather(invv, [jnp.full((16,), r, jnp.int32)])
            outv[r, :] = (p0v[r, :] + p1v[r, :]) * iv + nav[r, :]
            return carry

        lax.fori_loop(0, rows_w, row, 0)
        pltpu.sync_copy(outv, out_hbm.at[pl.ds(base, rows_w)])

    return combine_kernel


def kernel(node_attr, edge_index, edge_attr, edge_sh, W1, b1, W2, b2):
    n_nodes, d = node_attr.shape
    e = edge_attr.shape[0]
    n_pad = ((n_nodes + 1 + NW * D - 1) // (NW * D)) * (NW * D)

    src2 = edge_index[0].astype(jnp.int32).reshape(e // CHUNK, CHUNK)
    dst2 = edge_index[1].astype(jnp.int32).reshape(e // CHUNK, CHUNK)
    na_pad = jnp.concatenate(
        [node_attr, jnp.zeros((n_pad - n_nodes, d), jnp.float32)])

    x = _make_gather(e, n_nodes)(dst2, node_attr)
    tp = _run_tp(edge_attr.T, x, edge_sh.T, W1.T, b1[:, None],
                 W2.T.astype(jnp.bfloat16), b2[:, None])
    psum, pcnt = _make_scatter(e, n_pad)(src2, tp)
    out_pad = _make_combine(n_pad)(psum, pcnt, na_pad)
    return out_pad[:n_nodes]
